# baseline (speedup 1.0000x reference)
import jax
import jax.numpy as jnp
from jax.experimental import pallas as pl
from jax.experimental.pallas import tpu as pltpu

_B, _T, _D = 64, 32, 2048
_NRE = 7
_DFF = 2048
_TILE = 4
_PT = 21
_SLOTS = _PT * _TILE
_NWC = 4
_WC = _D // _NWC


def _gate_body(x_ref, wg_ref, scores_ref, route_ref, loss_ref,
               sxs_ref, dsts_ref, wslot_ref, te_ref,
               isf_ref, wb_ref, ne_ref, hn_ref):
    x = x_ref[...]
    xa = jnp.mean(x, axis=1)
    logits = jax.lax.dot_general(
        xa, wg_ref[...], (((1,), (1,)), ((), ())),
        preferred_element_type=jnp.float32)
    mx = jnp.max(logits, axis=1, keepdims=True)
    exl = jnp.exp(logits - mx)
    prob = exl / jnp.sum(exl, axis=1, keepdims=True)

    topv = jnp.max(prob, axis=1, keepdims=True)
    c7 = jax.lax.broadcasted_iota(jnp.int32, (_B, _NRE), 1)
    e2 = jnp.min(jnp.where(prob == topv, c7, _NRE), axis=1,
                 keepdims=True).astype(jnp.int32)

    imp = jnp.sum(prob, axis=0, keepdims=True)
    mu = jnp.mean(imp)
    var = jnp.sum((imp - mu) ** 2) / (_NRE - 1)
    loss_ref[...] = jnp.reshape(var / (mu * mu), (1, 1))

    onehot_e = (c7 == e2).astype(jnp.float32)
    counts = jnp.sum(onehot_e, axis=0, keepdims=True)
    r7 = jax.lax.broadcasted_iota(jnp.int32, (_NRE, _NRE), 0)
    s7 = jax.lax.broadcasted_iota(jnp.int32, (_NRE, _NRE), 1)
    lt7 = (r7 < s7).astype(jnp.float32)
    offs = jax.lax.dot_general(
        counts, lt7, (((1,), (0,)), ((), ())),
        preferred_element_type=jnp.float32)
    off_row = jnp.sum(onehot_e * offs, axis=1, keepdims=True)
    eq = jax.lax.dot_general(
        onehot_e, onehot_e, (((1,), (1,)), ((), ())),
        preferred_element_type=jnp.float32)
    ii = jax.lax.broadcasted_iota(jnp.int32, (_B, _B), 0)
    jj = jax.lax.broadcasted_iota(jnp.int32, (_B, _B), 1)
    ltmask = (jj < ii).astype(jnp.float32)
    rank_in = jnp.sum(eq * ltmask, axis=1, keepdims=True)
    pos = (off_row + rank_in).astype(jnp.int32)

    mf = (pos == jj).astype(jnp.float32)
    iif = ii.astype(jnp.float32)
    src = jnp.sum(mf * iif, axis=0, keepdims=True)

    half = (jj == (ii // 2)).astype(jnp.float32)
    p2 = jax.lax.dot_general(
        half, prob, (((1,), (0,)), ((), ())),
        preferred_element_type=jnp.float32)
    wrow = jnp.sum(p2 * onehot_e, axis=1, keepdims=True)
    wsrt = jnp.sum(mf * wrow, axis=0, keepdims=True)

    ntiles = jnp.floor((counts + (_TILE - 1)) / _TILE)
    to = jax.lax.dot_general(
        ntiles, lt7, (((1,), (0,)), ((), ())),
        preferred_element_type=jnp.float32)
    e_row = jax.lax.broadcasted_iota(jnp.int32, (1, _NRE), 1).astype(jnp.float32)
    emax = jnp.max(jnp.where(counts > 0, e_row, -1.0))

    scol = jax.lax.broadcasted_iota(jnp.int32, (_SLOTS, 1), 0).astype(jnp.float32)
    so = to * _TILE
    so_next = so + ntiles * _TILE
    esel = ((scol >= so) & (scol < so_next)).astype(jnp.float32)
    inrange = jnp.sum(esel, axis=1, keepdims=True)
    r_rel = scol - so
    qcand = offs + jnp.minimum(r_rel, counts - 1.0)
    q = jnp.sum(esel * qcand, axis=1, keepdims=True) + (1.0 - inrange) * (_B - 1)
    qi = q.astype(jnp.int32)
    jslot = jax.lax.broadcasted_iota(jnp.int32, (_SLOTS, _B), 1)
    g = (qi == jslot).astype(jnp.float32)
    src_slot = jnp.sum(g * src, axis=1, keepdims=True)
    w_slot = jnp.sum(g * wsrt, axis=1, keepdims=True)

    tcol = jax.lax.broadcasted_iota(jnp.int32, (_PT, 1), 0).astype(jnp.float32)
    tsel = ((tcol >= to) & (tcol < to + ntiles)).astype(jnp.float32)
    in_t = jnp.sum(tsel, axis=1, keepdims=True)
    te = jnp.sum(tsel * e_row, axis=1, keepdims=True) + (1.0 - in_t) * emax

    used = ntiles > 0.0
    started = (used & (to <= tcol)).astype(jnp.float32)
    runcnt = jnp.sum(started, axis=1, keepdims=True)
    wb = jnp.mod(runcnt - 1.0, 2.0)
    isf = jnp.sum((used & (to == tcol)).astype(jnp.float32), axis=1,
                  keepdims=True)
    candn = jnp.where(used & (e_row > te), e_row, float(_NRE))
    nexte = jnp.min(candn, axis=1, keepdims=True)
    hasn = (nexte < float(_NRE)).astype(jnp.float32)
    ne_cl = jnp.minimum(nexte, float(_NRE - 1))

    scores_ref[...] = topv
    route_ref[...] = e2
    dsts_ref[...] = src_slot.astype(jnp.int32)
    sxs_ref[...] = dsts_ref[...] // 2
    wslot_ref[...] = w_slot
    te_ref[...] = te.astype(jnp.int32)
    isf_ref[...] = isf.astype(jnp.int32)
    wb_ref[...] = wb.astype(jnp.int32)
    ne_ref[...] = ne_cl.astype(jnp.int32)
    hn_ref[...] = hasn.astype(jnp.int32)


def _up_body(sx_ref, te_ref, isf_ref, wb_ref, ne_ref, hn_ref,
             x_ref, w1_ref, b1_ref, h_ref, xsc_ref, wbuf_ref,
             sem_ref, wsem_ref):
    t = pl.program_id(0)
    cur = jax.lax.rem(t, 2)
    nxt = jax.lax.rem(t + 1, 2)
    b = wb_ref[t, 0]

    def row_copy(tt, buf, r):
        idx = sx_ref[tt * _TILE + r, 0]
        return pltpu.make_async_copy(
            x_ref.at[idx], xsc_ref.at[buf, pl.ds(r * _T, _T), :],
            sem_ref.at[buf, r])

    def w_copies(e, buf):
        return [pltpu.make_async_copy(
            w1_ref.at[e, pl.ds(c * _WC, _WC), :],
            wbuf_ref.at[buf, pl.ds(c * _WC, _WC), :],
            wsem_ref.at[buf, c]) for c in range(_NWC)]

    @pl.when(t == 0)
    def _():
        for cp in w_copies(te_ref[0, 0], b):
            cp.start()
        for r in range(_TILE):
            row_copy(t, cur, r).start()

    first = isf_ref[t, 0] == 1

    @pl.when(first & (hn_ref[t, 0] == 1))
    def _():
        for cp in w_copies(ne_ref[t, 0], 1 - b):
            cp.start()

    @pl.when(t + 1 < _PT)
    def _():
        for r in range(_TILE):
            row_copy(t + 1, nxt, r).start()

    @pl.when(first)
    def _():
        for cp in w_copies(te_ref[t, 0], b):
            cp.wait()

    for r in range(_TILE):
        row_copy(t, cur, r).wait()

    h = jnp.dot(xsc_ref[cur], wbuf_ref[b],
                preferred_element_type=jnp.float32) + b1_ref[0, 0]
    h_ref[0] = jax.nn.gelu(h)


def _down_body(dst_ref, te_ref, isf_ref, wb_ref, ne_ref, hn_ref,
               h_ref, w2_ref, b2_ref, wv_ref, out_ref,
               ysc_ref, wbuf_ref, sem_ref, wsem_ref):
    t = pl.program_id(0)
    cur = jax.lax.rem(t, 2)
    b = wb_ref[t, 0]

    def row_copy(tt, buf, r):
        idx = dst_ref[tt * _TILE + r, 0]
        return pltpu.make_async_copy(
            ysc_ref.at[buf, pl.ds(r * _T, _T), :], out_ref.at[idx],
            sem_ref.at[buf, r])

    def w_copies(e, buf):
        return [pltpu.make_async_copy(
            w2_ref.at[e, pl.ds(c * _WC, _WC), :],
            wbuf_ref.at[buf, pl.ds(c * _WC, _WC), :],
            wsem_ref.at[buf, c]) for c in range(_NWC)]

    @pl.when(t == 0)
    def _():
        for cp in w_copies(te_ref[0, 0], b):
            cp.start()

    first = isf_ref[t, 0] == 1

    @pl.when(first & (hn_ref[t, 0] == 1))
    def _():
        for cp in w_copies(ne_ref[t, 0], 1 - b):
            cp.start()

    @pl.when(t >= 2)
    def _():
        for r in range(_TILE):
            row_copy(t - 2, cur, r).wait()

    @pl.when(first)
    def _():
        for cp in w_copies(te_ref[t, 0], b):
            cp.wait()

    y = jnp.dot(h_ref[0], wbuf_ref[b],
                preferred_element_type=jnp.float32) + b2_ref[0, 0]
    for r in range(_TILE):
        ysc_ref[cur, pl.ds(r * _T, _T), :] = (
            y[r * _T:(r + 1) * _T, :] * wv_ref[t * _TILE + r, 0])
    for r in range(_TILE):
        row_copy(t, cur, r).start()

    @pl.when(t == _PT - 1)
    def _():
        for r in range(_TILE):
            row_copy(t - 1, jax.lax.rem(t + 1, 2), r).wait()
        for r in range(_TILE):
            row_copy(t, cur, r).wait()


def kernel(x, Wg, W1, b1, W2, b2):
    gate_out_shapes = (
        jax.ShapeDtypeStruct((_B, 1), jnp.float32),
        jax.ShapeDtypeStruct((_B, 1), jnp.int32),
        jax.ShapeDtypeStruct((1, 1), jnp.float32),
        jax.ShapeDtypeStruct((_SLOTS, 1), jnp.int32),
        jax.ShapeDtypeStruct((_SLOTS, 1), jnp.int32),
        jax.ShapeDtypeStruct((_SLOTS, 1), jnp.float32),
        jax.ShapeDtypeStruct((_PT, 1), jnp.int32),
        jax.ShapeDtypeStruct((_PT, 1), jnp.int32),
        jax.ShapeDtypeStruct((_PT, 1), jnp.int32),
        jax.ShapeDtypeStruct((_PT, 1), jnp.int32),
        jax.ShapeDtypeStruct((_PT, 1), jnp.int32),
    )
    (scores, route, loss, sxs, dsts, wslot, te,
     isf, wb, ne, hn) = pl.pallas_call(
        _gate_body, out_shape=gate_out_shapes)(x, Wg)

    h = pl.pallas_call(
        _up_body,
        grid_spec=pltpu.PrefetchScalarGridSpec(
            num_scalar_prefetch=6,
            grid=(_PT,),
            in_specs=[
                pl.BlockSpec(memory_space=pltpu.MemorySpace.HBM),
                pl.BlockSpec(memory_space=pltpu.MemorySpace.HBM),
                pl.BlockSpec((1, 1, _DFF),
                             lambda t, sx, te_, i_, w_, n_, h_: (te_[t, 0], 0, 0)),
            ],
            out_specs=pl.BlockSpec((1, _TILE * _T, _DFF),
                                   lambda t, sx, te_, i_, w_, n_, h_: (t, 0, 0)),
            scratch_shapes=[
                pltpu.VMEM((2, _TILE * _T, _D), jnp.float32),
                pltpu.VMEM((2, _D, _DFF), jnp.float32),
                pltpu.SemaphoreType.DMA((2, _TILE)),
                pltpu.SemaphoreType.DMA((2, _NWC)),
            ],
        ),
        out_shape=jax.ShapeDtypeStruct((_PT, _TILE * _T, _DFF), jnp.float32),
        compiler_params=pltpu.CompilerParams(
            dimension_semantics=("arbitrary",)),
    )(sxs, te, isf, wb, ne, hn, x, W1, b1.reshape(_NRE, 1, _DFF))

    out = pl.pallas_call(
        _down_body,
        grid_spec=pltpu.PrefetchScalarGridSpec(
            num_scalar_prefetch=6,
            grid=(_PT,),
            in_specs=[
                pl.BlockSpec((1, _TILE * _T, _DFF),
                             lambda t, dst, te_, i_, w_, n_, h_: (t, 0, 0)),
                pl.BlockSpec(memory_space=pltpu.MemorySpace.HBM),
                pl.BlockSpec((1, 1, _D),
                             lambda t, dst, te_, i_, w_, n_, h_: (te_[t, 0], 0, 0)),
                pl.BlockSpec(memory_space=pltpu.SMEM),
            ],
            out_specs=pl.BlockSpec(memory_space=pltpu.MemorySpace.HBM),
            scratch_shapes=[
                pltpu.VMEM((2, _TILE * _T, _D), jnp.float32),
                pltpu.VMEM((2, _DFF, _D), jnp.float32),
                pltpu.SemaphoreType.DMA((2, _TILE)),
                pltpu.SemaphoreType.DMA((2, _NWC)),
            ],
        ),
        out_shape=jax.ShapeDtypeStruct((_B, _T, _D), jnp.float32),
        compiler_params=pltpu.CompilerParams(
            dimension_semantics=("arbitrary",)),
    )(dsts, te, isf, wb, ne, hn, h, W2, b2.reshape(_NRE, 1, _D), wslot)

    return (out, scores.reshape(_B), route, jnp.arange(_B, dtype=jnp.int32),
            loss[0, 0])


# speedup vs baseline: 1.0716x; 1.0716x over previous
import jax
import jax.numpy as jnp
from jax.experimental import pallas as pl
from jax.experimental.pallas import tpu as pltpu

_B, _T, _D = 64, 32, 2048
_NRE = 7
_DFF = 2048
_TILE = 4
_PT = 21
_SLOTS = _PT * _TILE


def _gate_body(x_ref, wg_ref, scores_ref, route_ref, loss_ref,
               sxs_ref, dsts_ref, wslot_ref, te_ref,
               isf_ref, wb_ref, ne_ref, hn_ref):
    x = x_ref[...]
    xa = jnp.mean(x, axis=1)
    logits = jax.lax.dot_general(
        xa, wg_ref[...], (((1,), (1,)), ((), ())),
        preferred_element_type=jnp.float32)
    mx = jnp.max(logits, axis=1, keepdims=True)
    exl = jnp.exp(logits - mx)
    prob = exl / jnp.sum(exl, axis=1, keepdims=True)

    topv = jnp.max(prob, axis=1, keepdims=True)
    c7 = jax.lax.broadcasted_iota(jnp.int32, (_B, _NRE), 1)
    e2 = jnp.min(jnp.where(prob == topv, c7, _NRE), axis=1,
                 keepdims=True).astype(jnp.int32)

    imp = jnp.sum(prob, axis=0, keepdims=True)
    mu = jnp.mean(imp)
    var = jnp.sum((imp - mu) ** 2) / (_NRE - 1)
    loss_ref[...] = jnp.reshape(var / (mu * mu), (1, 1))

    onehot_e = (c7 == e2).astype(jnp.float32)
    counts = jnp.sum(onehot_e, axis=0, keepdims=True)
    r7 = jax.lax.broadcasted_iota(jnp.int32, (_NRE, _NRE), 0)
    s7 = jax.lax.broadcasted_iota(jnp.int32, (_NRE, _NRE), 1)
    lt7 = (r7 < s7).astype(jnp.float32)
    offs = jax.lax.dot_general(
        counts, lt7, (((1,), (0,)), ((), ())),
        preferred_element_type=jnp.float32)
    off_row = jnp.sum(onehot_e * offs, axis=1, keepdims=True)
    eq = jax.lax.dot_general(
        onehot_e, onehot_e, (((1,), (1,)), ((), ())),
        preferred_element_type=jnp.float32)
    ii = jax.lax.broadcasted_iota(jnp.int32, (_B, _B), 0)
    jj = jax.lax.broadcasted_iota(jnp.int32, (_B, _B), 1)
    ltmask = (jj < ii).astype(jnp.float32)
    rank_in = jnp.sum(eq * ltmask, axis=1, keepdims=True)
    pos = (off_row + rank_in).astype(jnp.int32)

    mf = (pos == jj).astype(jnp.float32)
    iif = ii.astype(jnp.float32)
    src = jnp.sum(mf * iif, axis=0, keepdims=True)

    half = (jj == (ii // 2)).astype(jnp.float32)
    p2 = jax.lax.dot_general(
        half, prob, (((1,), (0,)), ((), ())),
        preferred_element_type=jnp.float32)
    wrow = jnp.sum(p2 * onehot_e, axis=1, keepdims=True)
    wsrt = jnp.sum(mf * wrow, axis=0, keepdims=True)

    ntiles = jnp.floor((counts + (_TILE - 1)) / _TILE)
    to = jax.lax.dot_general(
        ntiles, lt7, (((1,), (0,)), ((), ())),
        preferred_element_type=jnp.float32)
    e_row = jax.lax.broadcasted_iota(jnp.int32, (1, _NRE), 1).astype(jnp.float32)
    emax = jnp.max(jnp.where(counts > 0, e_row, -1.0))

    scol = jax.lax.broadcasted_iota(jnp.int32, (_SLOTS, 1), 0).astype(jnp.float32)
    so = to * _TILE
    so_next = so + ntiles * _TILE
    esel = ((scol >= so) & (scol < so_next)).astype(jnp.float32)
    inrange = jnp.sum(esel, axis=1, keepdims=True)
    r_rel = scol - so
    qcand = offs + jnp.minimum(r_rel, counts - 1.0)
    q = jnp.sum(esel * qcand, axis=1, keepdims=True) + (1.0 - inrange) * (_B - 1)
    qi = q.astype(jnp.int32)
    jslot = jax.lax.broadcasted_iota(jnp.int32, (_SLOTS, _B), 1)
    g = (qi == jslot).astype(jnp.float32)
    src_slot = jnp.sum(g * src, axis=1, keepdims=True)
    w_slot = jnp.sum(g * wsrt, axis=1, keepdims=True)

    tcol = jax.lax.broadcasted_iota(jnp.int32, (_PT, 1), 0).astype(jnp.float32)
    tsel = ((tcol >= to) & (tcol < to + ntiles)).astype(jnp.float32)
    in_t = jnp.sum(tsel, axis=1, keepdims=True)
    te = jnp.sum(tsel * e_row, axis=1, keepdims=True) + (1.0 - in_t) * emax

    used = ntiles > 0.0
    started = (used & (to <= tcol)).astype(jnp.float32)
    runcnt = jnp.sum(started, axis=1, keepdims=True)
    wb = jnp.mod(runcnt - 1.0, 2.0)
    isf = jnp.sum((used & (to == tcol)).astype(jnp.float32), axis=1,
                  keepdims=True)
    candn = jnp.where(used & (e_row > te), e_row, float(_NRE))
    nexte = jnp.min(candn, axis=1, keepdims=True)
    hasn = (nexte < float(_NRE)).astype(jnp.float32)
    ne_cl = jnp.minimum(nexte, float(_NRE - 1))

    scores_ref[...] = topv
    route_ref[...] = e2
    dsts_ref[...] = src_slot.astype(jnp.int32)
    sxs_ref[...] = dsts_ref[...] // 2
    wslot_ref[...] = w_slot
    te_ref[...] = te.astype(jnp.int32)
    isf_ref[...] = isf.astype(jnp.int32)
    wb_ref[...] = wb.astype(jnp.int32)
    ne_ref[...] = ne_cl.astype(jnp.int32)
    hn_ref[...] = hasn.astype(jnp.int32)


def _up_body(sx_ref, te_ref, isf_ref, wb_ref, ne_ref, hn_ref,
             x_ref, w1_ref, b1_ref, h_ref, xsc_ref, wbuf_ref,
             sem_ref, wsem_ref):
    t = pl.program_id(0)
    cur = jax.lax.rem(t, 2)
    nxt = jax.lax.rem(t + 1, 2)
    b = wb_ref[t, 0]

    def row_copy(tt, buf, r):
        idx = sx_ref[tt * _TILE + r, 0]
        return pltpu.make_async_copy(
            x_ref.at[idx], xsc_ref.at[buf, pl.ds(r * _T, _T), :],
            sem_ref.at[buf, r])

    def w_copy(e, buf):
        return pltpu.make_async_copy(
            w1_ref.at[e], wbuf_ref.at[buf], wsem_ref.at[buf])

    @pl.when(t == 0)
    def _():
        w_copy(te_ref[0, 0], b).start()
        for r in range(_TILE):
            row_copy(t, cur, r).start()

    first = isf_ref[t, 0] == 1

    @pl.when(first & (hn_ref[t, 0] == 1))
    def _():
        w_copy(ne_ref[t, 0], 1 - b).start()

    @pl.when(t + 1 < _PT)
    def _():
        for r in range(_TILE):
            row_copy(t + 1, nxt, r).start()

    @pl.when(first)
    def _():
        w_copy(te_ref[t, 0], b).wait()

    for r in range(_TILE):
        row_copy(t, cur, r).wait()

    h = jnp.dot(xsc_ref[cur], wbuf_ref[b],
                preferred_element_type=jnp.float32) + b1_ref[0, 0]
    h_ref[0] = jax.nn.gelu(h).astype(jnp.bfloat16)


def _down_body(dst_ref, te_ref, isf_ref, wb_ref, ne_ref, hn_ref,
               h_ref, w2_ref, b2_ref, wv_ref, out_ref,
               ysc_ref, wbuf_ref, sem_ref, wsem_ref):
    t = pl.program_id(0)
    cur = jax.lax.rem(t, 2)
    b = wb_ref[t, 0]

    def row_copy(tt, buf, r):
        idx = dst_ref[tt * _TILE + r, 0]
        return pltpu.make_async_copy(
            ysc_ref.at[buf, pl.ds(r * _T, _T), :], out_ref.at[idx],
            sem_ref.at[buf, r])

    def w_copy(e, buf):
        return pltpu.make_async_copy(
            w2_ref.at[e], wbuf_ref.at[buf], wsem_ref.at[buf])

    @pl.when(t == 0)
    def _():
        w_copy(te_ref[0, 0], b).start()

    first = isf_ref[t, 0] == 1

    @pl.when(first & (hn_ref[t, 0] == 1))
    def _():
        w_copy(ne_ref[t, 0], 1 - b).start()

    @pl.when(t >= 2)
    def _():
        for r in range(_TILE):
            row_copy(t - 2, cur, r).wait()

    @pl.when(first)
    def _():
        w_copy(te_ref[t, 0], b).wait()

    y = jnp.dot(h_ref[0].astype(jnp.float32), wbuf_ref[b],
                preferred_element_type=jnp.float32) + b2_ref[0, 0]
    for r in range(_TILE):
        ysc_ref[cur, pl.ds(r * _T, _T), :] = (
            y[r * _T:(r + 1) * _T, :] * wv_ref[t * _TILE + r, 0])
    for r in range(_TILE):
        row_copy(t, cur, r).start()

    @pl.when(t == _PT - 1)
    def _():
        for r in range(_TILE):
            row_copy(t - 1, jax.lax.rem(t + 1, 2), r).wait()
        for r in range(_TILE):
            row_copy(t, cur, r).wait()


def kernel(x, Wg, W1, b1, W2, b2):
    gate_out_shapes = (
        jax.ShapeDtypeStruct((_B, 1), jnp.float32),
        jax.ShapeDtypeStruct((_B, 1), jnp.int32),
        jax.ShapeDtypeStruct((1, 1), jnp.float32),
        jax.ShapeDtypeStruct((_SLOTS, 1), jnp.int32),
        jax.ShapeDtypeStruct((_SLOTS, 1), jnp.int32),
        jax.ShapeDtypeStruct((_SLOTS, 1), jnp.float32),
        jax.ShapeDtypeStruct((_PT, 1), jnp.int32),
        jax.ShapeDtypeStruct((_PT, 1), jnp.int32),
        jax.ShapeDtypeStruct((_PT, 1), jnp.int32),
        jax.ShapeDtypeStruct((_PT, 1), jnp.int32),
        jax.ShapeDtypeStruct((_PT, 1), jnp.int32),
    )
    (scores, route, loss, sxs, dsts, wslot, te,
     isf, wb, ne, hn) = pl.pallas_call(
        _gate_body, out_shape=gate_out_shapes)(x, Wg)

    h = pl.pallas_call(
        _up_body,
        grid_spec=pltpu.PrefetchScalarGridSpec(
            num_scalar_prefetch=6,
            grid=(_PT,),
            in_specs=[
                pl.BlockSpec(memory_space=pltpu.MemorySpace.HBM),
                pl.BlockSpec(memory_space=pltpu.MemorySpace.HBM),
                pl.BlockSpec((1, 1, _DFF),
                             lambda t, sx, te_, i_, w_, n_, h_: (te_[t, 0], 0, 0)),
            ],
            out_specs=pl.BlockSpec((1, _TILE * _T, _DFF),
                                   lambda t, sx, te_, i_, w_, n_, h_: (t, 0, 0)),
            scratch_shapes=[
                pltpu.VMEM((2, _TILE * _T, _D), jnp.float32),
                pltpu.VMEM((2, _D, _DFF), jnp.float32),
                pltpu.SemaphoreType.DMA((2, _TILE)),
                pltpu.SemaphoreType.DMA((2,)),
            ],
        ),
        out_shape=jax.ShapeDtypeStruct((_PT, _TILE * _T, _DFF), jnp.bfloat16),
        compiler_params=pltpu.CompilerParams(
            dimension_semantics=("arbitrary",)),
    )(sxs, te, isf, wb, ne, hn, x, W1, b1.reshape(_NRE, 1, _DFF))

    out = pl.pallas_call(
        _down_body,
        grid_spec=pltpu.PrefetchScalarGridSpec(
            num_scalar_prefetch=6,
            grid=(_PT,),
            in_specs=[
                pl.BlockSpec((1, _TILE * _T, _DFF),
                             lambda t, dst, te_, i_, w_, n_, h_: (t, 0, 0)),
                pl.BlockSpec(memory_space=pltpu.MemorySpace.HBM),
                pl.BlockSpec((1, 1, _D),
                             lambda t, dst, te_, i_, w_, n_, h_: (te_[t, 0], 0, 0)),
                pl.BlockSpec(memory_space=pltpu.SMEM),
            ],
            out_specs=pl.BlockSpec(memory_space=pltpu.MemorySpace.HBM),
            scratch_shapes=[
                pltpu.VMEM((2, _TILE * _T, _D), jnp.float32),
                pltpu.VMEM((2, _DFF, _D), jnp.float32),
                pltpu.SemaphoreType.DMA((2, _TILE)),
                pltpu.SemaphoreType.DMA((2,)),
            ],
        ),
        out_shape=jax.ShapeDtypeStruct((_B, _T, _D), jnp.float32),
        compiler_params=pltpu.CompilerParams(
            dimension_semantics=("arbitrary",)),
    )(dsts, te, isf, wb, ne, hn, h, W2, b2.reshape(_NRE, 1, _D), wslot)

    return (out, scores.reshape(_B), route, jnp.arange(_B, dtype=jnp.int32),
            loss[0, 0])


# speedup vs baseline: 1.2215x; 1.1399x over previous
import jax
import jax.numpy as jnp
from jax.experimental import pallas as pl
from jax.experimental.pallas import tpu as pltpu

_B, _T, _D = 64, 32, 2048
_NRE = 7
_DFF = 2048
_TILE = 4
_PT = 21
_SLOTS = _PT * _TILE


def _gate_body(x_ref, wg_ref, scores_ref, route_ref, loss_ref,
               sxs_ref, dsts_ref, wslot_ref, te_ref,
               isf_ref, wb_ref, ne_ref, hn_ref, il_ref):
    x = x_ref[...]
    xa = jnp.mean(x, axis=1)
    logits = jax.lax.dot_general(
        xa, wg_ref[...], (((1,), (1,)), ((), ())),
        preferred_element_type=jnp.float32)
    mx = jnp.max(logits, axis=1, keepdims=True)
    exl = jnp.exp(logits - mx)
    prob = exl / jnp.sum(exl, axis=1, keepdims=True)

    topv = jnp.max(prob, axis=1, keepdims=True)
    c7 = jax.lax.broadcasted_iota(jnp.int32, (_B, _NRE), 1)
    e2 = jnp.min(jnp.where(prob == topv, c7, _NRE), axis=1,
                 keepdims=True).astype(jnp.int32)

    imp = jnp.sum(prob, axis=0, keepdims=True)
    mu = jnp.mean(imp)
    var = jnp.sum((imp - mu) ** 2) / (_NRE - 1)
    loss_ref[...] = jnp.reshape(var / (mu * mu), (1, 1))

    onehot_e = (c7 == e2).astype(jnp.float32)
    counts = jnp.sum(onehot_e, axis=0, keepdims=True)
    r7 = jax.lax.broadcasted_iota(jnp.int32, (_NRE, _NRE), 0)
    s7 = jax.lax.broadcasted_iota(jnp.int32, (_NRE, _NRE), 1)
    lt7 = (r7 < s7).astype(jnp.float32)
    offs = jax.lax.dot_general(
        counts, lt7, (((1,), (0,)), ((), ())),
        preferred_element_type=jnp.float32)
    off_row = jnp.sum(onehot_e * offs, axis=1, keepdims=True)
    eq = jax.lax.dot_general(
        onehot_e, onehot_e, (((1,), (1,)), ((), ())),
        preferred_element_type=jnp.float32)
    ii = jax.lax.broadcasted_iota(jnp.int32, (_B, _B), 0)
    jj = jax.lax.broadcasted_iota(jnp.int32, (_B, _B), 1)
    ltmask = (jj < ii).astype(jnp.float32)
    rank_in = jnp.sum(eq * ltmask, axis=1, keepdims=True)
    pos = (off_row + rank_in).astype(jnp.int32)

    mf = (pos == jj).astype(jnp.float32)
    iif = ii.astype(jnp.float32)
    src = jnp.sum(mf * iif, axis=0, keepdims=True)

    half = (jj == (ii // 2)).astype(jnp.float32)
    p2 = jax.lax.dot_general(
        half, prob, (((1,), (0,)), ((), ())),
        preferred_element_type=jnp.float32)
    wrow = jnp.sum(p2 * onehot_e, axis=1, keepdims=True)
    wsrt = jnp.sum(mf * wrow, axis=0, keepdims=True)

    ntiles = jnp.floor((counts + (_TILE - 1)) / _TILE)
    to = jax.lax.dot_general(
        ntiles, lt7, (((1,), (0,)), ((), ())),
        preferred_element_type=jnp.float32)
    e_row = jax.lax.broadcasted_iota(jnp.int32, (1, _NRE), 1).astype(jnp.float32)
    emax = jnp.max(jnp.where(counts > 0, e_row, -1.0))

    scol = jax.lax.broadcasted_iota(jnp.int32, (_SLOTS, 1), 0).astype(jnp.float32)
    so = to * _TILE
    so_next = so + ntiles * _TILE
    esel = ((scol >= so) & (scol < so_next)).astype(jnp.float32)
    inrange = jnp.sum(esel, axis=1, keepdims=True)
    r_rel = scol - so
    qcand = offs + jnp.minimum(r_rel, counts - 1.0)
    q = jnp.sum(esel * qcand, axis=1, keepdims=True) + (1.0 - inrange) * (_B - 1)
    qi = q.astype(jnp.int32)
    jslot = jax.lax.broadcasted_iota(jnp.int32, (_SLOTS, _B), 1)
    g = (qi == jslot).astype(jnp.float32)
    src_slot = jnp.sum(g * src, axis=1, keepdims=True)
    w_slot = jnp.sum(g * wsrt, axis=1, keepdims=True)

    tcol = jax.lax.broadcasted_iota(jnp.int32, (_PT, 1), 0).astype(jnp.float32)
    tsel = ((tcol >= to) & (tcol < to + ntiles)).astype(jnp.float32)
    in_t = jnp.sum(tsel, axis=1, keepdims=True)
    te = jnp.sum(tsel * e_row, axis=1, keepdims=True) + (1.0 - in_t) * emax

    used = ntiles > 0.0
    started = (used & (to <= tcol)).astype(jnp.float32)
    runcnt = jnp.sum(started, axis=1, keepdims=True)
    wb = jnp.mod(runcnt - 1.0, 2.0)
    isf = jnp.sum((used & (to == tcol)).astype(jnp.float32), axis=1,
                  keepdims=True)
    il = jnp.sum((used & (to + ntiles - 1.0 == tcol)).astype(jnp.float32),
                 axis=1, keepdims=True)
    candn = jnp.where(used & (e_row > te), e_row, float(_NRE))
    nexte = jnp.min(candn, axis=1, keepdims=True)
    hasn = (nexte < float(_NRE)).astype(jnp.float32)
    ne_cl = jnp.minimum(nexte, float(_NRE - 1))

    scores_ref[...] = topv
    route_ref[...] = e2
    dsts_ref[...] = src_slot.astype(jnp.int32)
    sxs_ref[...] = dsts_ref[...] // 2
    wslot_ref[...] = w_slot
    te_ref[...] = te.astype(jnp.int32)
    isf_ref[...] = isf.astype(jnp.int32)
    wb_ref[...] = wb.astype(jnp.int32)
    ne_ref[...] = ne_cl.astype(jnp.int32)
    hn_ref[...] = hasn.astype(jnp.int32)
    il_ref[...] = il.astype(jnp.int32)


def _ffn_body(sx_ref, dst_ref, te_ref, isf_ref, wb_ref, ne_ref, hn_ref,
              il_ref, x_ref, w1_ref, b1_ref, w2_ref, b2_ref, wv_ref, out_ref,
              xsc_ref, w1buf_ref, w2buf_ref, hsc_ref, ysc_ref,
              xsem_ref, w1sem_ref, w2sem_ref, ysem_ref):
    t = pl.program_id(0)
    cur = jax.lax.rem(t, 2)
    nxt = jax.lax.rem(t + 1, 2)
    b = wb_ref[t, 0]

    def gather_copy(tt, buf, r):
        idx = sx_ref[tt * _TILE + r, 0]
        return pltpu.make_async_copy(
            x_ref.at[idx], xsc_ref.at[buf, pl.ds(r * _T, _T), :],
            xsem_ref.at[buf, r])

    def scatter_copy(tt, buf, r):
        idx = dst_ref[tt * _TILE + r, 0]
        return pltpu.make_async_copy(
            ysc_ref.at[buf, pl.ds(r * _T, _T), :], out_ref.at[idx],
            ysem_ref.at[buf, r])

    def w1_copy(e, buf):
        return pltpu.make_async_copy(
            w1_ref.at[e], w1buf_ref.at[buf], w1sem_ref.at[buf])

    def w2_copy(e):
        return pltpu.make_async_copy(
            w2_ref.at[e], w2buf_ref, w2sem_ref.at[0])

    @pl.when(t == 0)
    def _():
        w1_copy(te_ref[0, 0], b).start()
        w2_copy(te_ref[0, 0]).start()
        for r in range(_TILE):
            gather_copy(t, cur, r).start()

    first = isf_ref[t, 0] == 1

    @pl.when(first & (hn_ref[t, 0] == 1))
    def _():
        w1_copy(ne_ref[t, 0], 1 - b).start()

    @pl.when(t + 1 < _PT)
    def _():
        for r in range(_TILE):
            gather_copy(t + 1, nxt, r).start()

    @pl.when(t >= 2)
    def _():
        for r in range(_TILE):
            scatter_copy(t - 2, cur, r).wait()

    @pl.when(first)
    def _():
        w1_copy(te_ref[t, 0], b).wait()
        w2_copy(te_ref[t, 0]).wait()

    for r in range(_TILE):
        gather_copy(t, cur, r).wait()

    h = jnp.dot(xsc_ref[cur], w1buf_ref[b],
                preferred_element_type=jnp.float32) + b1_ref[0, 0]
    hsc_ref[...] = jax.nn.gelu(h)
    y = jnp.dot(hsc_ref[...], w2buf_ref[...],
                preferred_element_type=jnp.float32) + b2_ref[0, 0]
    for r in range(_TILE):
        ysc_ref[cur, pl.ds(r * _T, _T), :] = (
            y[r * _T:(r + 1) * _T, :] * wv_ref[t * _TILE + r, 0])
    for r in range(_TILE):
        scatter_copy(t, cur, r).start()

    @pl.when((il_ref[t, 0] == 1) & (hn_ref[t, 0] == 1))
    def _():
        w2_copy(ne_ref[t, 0]).start()

    @pl.when(t == _PT - 1)
    def _():
        for r in range(_TILE):
            scatter_copy(t - 1, jax.lax.rem(t + 1, 2), r).wait()
        for r in range(_TILE):
            scatter_copy(t, cur, r).wait()


def kernel(x, Wg, W1, b1, W2, b2):
    gate_out_shapes = (
        jax.ShapeDtypeStruct((_B, 1), jnp.float32),
        jax.ShapeDtypeStruct((_B, 1), jnp.int32),
        jax.ShapeDtypeStruct((1, 1), jnp.float32),
        jax.ShapeDtypeStruct((_SLOTS, 1), jnp.int32),
        jax.ShapeDtypeStruct((_SLOTS, 1), jnp.int32),
        jax.ShapeDtypeStruct((_SLOTS, 1), jnp.float32),
        jax.ShapeDtypeStruct((_PT, 1), jnp.int32),
        jax.ShapeDtypeStruct((_PT, 1), jnp.int32),
        jax.ShapeDtypeStruct((_PT, 1), jnp.int32),
        jax.ShapeDtypeStruct((_PT, 1), jnp.int32),
        jax.ShapeDtypeStruct((_PT, 1), jnp.int32),
        jax.ShapeDtypeStruct((_PT, 1), jnp.int32),
    )
    (scores, route, loss, sxs, dsts, wslot, te,
     isf, wb, ne, hn, il) = pl.pallas_call(
        _gate_body, out_shape=gate_out_shapes)(x, Wg)

    out = pl.pallas_call(
        _ffn_body,
        grid_spec=pltpu.PrefetchScalarGridSpec(
            num_scalar_prefetch=8,
            grid=(_PT,),
            in_specs=[
                pl.BlockSpec(memory_space=pltpu.MemorySpace.HBM),
                pl.BlockSpec(memory_space=pltpu.MemorySpace.HBM),
                pl.BlockSpec((1, 1, _DFF),
                             lambda t, *_: (_[2][t, 0], 0, 0)),
                pl.BlockSpec(memory_space=pltpu.MemorySpace.HBM),
                pl.BlockSpec((1, 1, _D),
                             lambda t, *_: (_[2][t, 0], 0, 0)),
                pl.BlockSpec(memory_space=pltpu.SMEM),
            ],
            out_specs=pl.BlockSpec(memory_space=pltpu.MemorySpace.HBM),
            scratch_shapes=[
                pltpu.VMEM((2, _TILE * _T, _D), jnp.float32),
                pltpu.VMEM((2, _D, _DFF), jnp.float32),
                pltpu.VMEM((_DFF, _D), jnp.float32),
                pltpu.VMEM((_TILE * _T, _DFF), jnp.float32),
                pltpu.VMEM((2, _TILE * _T, _D), jnp.float32),
                pltpu.SemaphoreType.DMA((2, _TILE)),
                pltpu.SemaphoreType.DMA((2,)),
                pltpu.SemaphoreType.DMA((1,)),
                pltpu.SemaphoreType.DMA((2, _TILE)),
            ],
        ),
        out_shape=jax.ShapeDtypeStruct((_B, _T, _D), jnp.float32),
        compiler_params=pltpu.CompilerParams(
            dimension_semantics=("arbitrary",)),
    )(sxs, dsts, te, isf, wb, ne, hn, il,
      x, W1, b1.reshape(_NRE, 1, _DFF), W2, b2.reshape(_NRE, 1, _D), wslot)

    return (out, scores.reshape(_B), route, jnp.arange(_B, dtype=jnp.int32),
            loss[0, 0])
